# baseline (speedup 1.0000x reference)
import functools

import jax
import jax.numpy as jnp
from jax import lax
from jax.experimental import pallas as pl
from jax.experimental.pallas import tpu as pltpu
from jax.experimental.pallas import tpu_sc as plsc

N = 10000
E = 320000
D = 128

NC = 2
NS = 16
DEGW = 16
CH = 96
ECT = -(-E // (CH * NS * NC))
E_EDGE = ECT * NS * NC * CH
DCH = 128
DCT = 2 * -(-E // (DCH * NS * NC * 2))
E_PAD = DCT * NS * NC * DCH
TRASH = N
N_PAD = 10112
RPT = N_PAD // NS

@functools.cache
def _sc_kernels():
    mesh = plsc.VectorSubcoreMesh(
        core_axis_name="c", subcore_axis_name="s", num_cores=NC, num_subcores=NS
    )
    deg = functools.partial(
        pl.kernel,
        out_type=jax.ShapeDtypeStruct((NC, N_PAD, D), jnp.float32),
        mesh=mesh,
        scratch_types=[
            pltpu.VMEM((DCH,), jnp.int32),
            pltpu.VMEM((DCH,), jnp.int32),
            pltpu.VMEM((DCH, D), jnp.float32),
            pltpu.SemaphoreType.DMA,
            pltpu.SemaphoreType.DMA,
            pltpu.VMEM_SHARED((N_PAD, D), jnp.float32),
        ],
    )(_deg_scatter_body)
    edge = functools.partial(
        pl.kernel,
        out_type=jax.ShapeDtypeStruct((NC, N_PAD, D), jnp.float32),
        mesh=mesh,
        scratch_types=(
            [pltpu.VMEM((CH,), jnp.int32) for _ in range(4)]
            + [pltpu.VMEM((CH,), jnp.int32) for _ in range(4)]
            + [pltpu.VMEM((CH, D), jnp.float32) for _ in range(4)]
            + [pltpu.SemaphoreType.DMA for _ in range(4)]
            + [pltpu.SemaphoreType.DMA for _ in range(4)]
            + [pltpu.VMEM_SHARED((N_PAD, D), jnp.float32)]
        ),
    )(_edge_scatter_body)
    return deg, edge


def _tile_chunk0(c, s, cpt):
    return (c * NS + s) * cpt


def _deg_scatter_body(col_hbm, ones_hbm, zeros_hbm, out_hbm,
                      cidx0, cidx1, ones_v, sem0, sem1, acc):
    c = lax.axis_index("c")
    s = lax.axis_index("s")
    rbase = pl.multiple_of(s * RPT, 8)
    pltpu.sync_copy(zeros_hbm.at[pl.ds(rbase, RPT)], acc.at[pl.ds(rbase, RPT)])
    pltpu.sync_copy(ones_hbm, ones_v)
    plsc.subcore_barrier()
    chunk0 = _tile_chunk0(c, s, DCT)
    m = DCT // 2

    def _eoff(q):
        return pl.multiple_of((chunk0 + q) * DCH, 8)

    pltpu.sync_copy(col_hbm.at[pl.ds(_eoff(0), DCH)], cidx0)

    def pair(p, carry):
        qa = 2 * p
        pltpu.async_copy(ones_v, acc.at[cidx0], sem0, add=True)
        pltpu.sync_copy(col_hbm.at[pl.ds(_eoff(qa + 1), DCH)], cidx1)
        pltpu.async_copy(ones_v, acc.at[cidx1], sem1, add=True)
        pltpu.make_async_copy(ones_v, acc.at[cidx0], sem0).wait()
        qn = jnp.minimum(qa + 2, DCT - 1)
        pltpu.sync_copy(col_hbm.at[pl.ds(_eoff(qn), DCH)], cidx0)
        pltpu.make_async_copy(ones_v, acc.at[cidx1], sem1).wait()
        return carry

    lax.fori_loop(0, m, pair, 0)
    plsc.subcore_barrier()
    pltpu.sync_copy(acc.at[pl.ds(rbase, RPT)], out_hbm.at[c, pl.ds(rbase, RPT)])


def _edge_scatter_body(g_hbm, row_hbm, col_hbm, zeros_hbm, out_hbm,
                       ridx0, ridx1, ridx2, ridx3,
                       cidx0, cidx1, cidx2, cidx3,
                       rows0, rows1, rows2, rows3,
                       semg0, semg1, semg2, semg3,
                       sems0, sems1, sems2, sems3, acc):
    c = lax.axis_index("c")
    s = lax.axis_index("s")
    rbase = pl.multiple_of(s * RPT, 8)
    pltpu.sync_copy(zeros_hbm.at[pl.ds(rbase, RPT)], acc.at[pl.ds(rbase, RPT)])
    plsc.subcore_barrier()
    chunk0 = _tile_chunk0(c, s, ECT)
    n = ECT

    ridx = (ridx0, ridx1, ridx2, ridx3)
    cidx = (cidx0, cidx1, cidx2, cidx3)
    rows = (rows0, rows1, rows2, rows3)
    semg = (semg0, semg1, semg2, semg3)
    sems = (sems0, sems1, sems2, sems3)

    def _eoff(q):
        return pl.multiple_of((chunk0 + q) * CH, 8)

    def _load_idx(q, b):
        off = _eoff(q)
        pltpu.sync_copy(row_hbm.at[pl.ds(off, CH)], ridx[b])
        pltpu.sync_copy(col_hbm.at[pl.ds(off, CH)], cidx[b])

    def _gather(b):
        pltpu.async_copy(g_hbm.at[ridx[b]], rows[b], semg[b])

    def _gather_wait(b):
        pltpu.make_async_copy(g_hbm.at[ridx[b]], rows[b], semg[b]).wait()

    def _scatter(b):
        pltpu.async_copy(rows[b], acc.at[cidx[b]], sems[b], add=True)

    def _scatter_wait(b):
        pltpu.make_async_copy(rows[b], acc.at[cidx[b]], sems[b]).wait()

    _load_idx(0, 0)
    _gather(0)
    _load_idx(1, 1)
    _gather(1)
    _gather_wait(0)
    _scatter(0)
    _load_idx(2, 2)
    _gather(2)
    _gather_wait(1)
    _scatter(1)
    _load_idx(3, 3)
    _gather(3)

    m = (n - 2) // 4

    def quad(p, carry):
        q = 2 + 4 * p
        for k, b in enumerate((2, 3, 0, 1)):
            _gather_wait(b)
            _scatter_wait((b + 2) % 4)
            _scatter(b)
            nb = (b + 2) % 4
            _load_idx(jnp.minimum(q + k + 2, n - 1), nb)
            _gather(nb)
        return carry

    lax.fori_loop(0, m, quad, 0)

    _scatter_wait(0)
    _scatter_wait(1)
    _gather_wait(2)
    _gather_wait(3)

    def tail(q, carry):
        _load_idx(q, 2)
        _gather(2)
        _gather_wait(2)
        _scatter(2)
        _scatter_wait(2)
        return carry

    lax.fori_loop(2 + 4 * m, n, tail, 0)
    plsc.subcore_barrier()
    pltpu.sync_copy(acc.at[pl.ds(rbase, RPT)], out_hbm.at[c, pl.ds(rbase, RPT)])


BLK = 2000


def _layer1_body(deg_ref, x_ref, w_ref, b_ref, g_ref, dis_ref):
    deg = deg_ref[0] + deg_ref[1]
    dis = jnp.where(deg > 0.0, lax.rsqrt(deg), 0.0)
    h = lax.dot_general(
        x_ref[...], w_ref[...], (((1,), (1,)), ((), ())),
        preferred_element_type=jnp.float32,
    ) + b_ref[...]
    g_ref[...] = h * dis[:, :1]
    dis_ref[...] = dis[:, :DEGW]


def _layer1(deg_parts, x, W1, b1):
    return pl.pallas_call(
        _layer1_body,
        grid=(N // BLK,),
        in_specs=[
            pl.BlockSpec((NC, BLK, D), lambda i: (0, i, 0)),
            pl.BlockSpec((BLK, D), lambda i: (i, 0)),
            pl.BlockSpec((D, D), lambda i: (0, 0)),
            pl.BlockSpec((1, D), lambda i: (0, 0)),
        ],
        out_specs=[
            pl.BlockSpec((BLK, D), lambda i: (i, 0)),
            pl.BlockSpec((BLK, DEGW), lambda i: (i, 0)),
        ],
        out_shape=[
            jax.ShapeDtypeStruct((N, D), jnp.float32),
            jax.ShapeDtypeStruct((N, DEGW), jnp.float32),
        ],
    )(deg_parts, x, W1, b1)


def _layer2_body(s_ref, dis_ref, w_ref, b_ref, g_ref):
    s = s_ref[0] + s_ref[1]
    out1 = s * dis_ref[...][:, :1]
    t = jnp.maximum(out1, 0.0)
    h = lax.dot_general(
        t, w_ref[...], (((1,), (1,)), ((), ())),
        preferred_element_type=jnp.float32,
    ) + b_ref[...]
    g_ref[...] = h * dis_ref[...][:, :1]


def _layer2(s_parts, dis, W2, b2):
    return pl.pallas_call(
        _layer2_body,
        grid=(N // BLK,),
        in_specs=[
            pl.BlockSpec((NC, BLK, D), lambda i: (0, i, 0)),
            pl.BlockSpec((BLK, DEGW), lambda i: (i, 0)),
            pl.BlockSpec((D, D), lambda i: (0, 0)),
            pl.BlockSpec((1, D), lambda i: (0, 0)),
        ],
        out_specs=pl.BlockSpec((BLK, D), lambda i: (i, 0)),
        out_shape=jax.ShapeDtypeStruct((N, D), jnp.float32),
    )(s_parts, dis, W2, b2)


def _finish_body(s_ref, dis_ref, out_ref):
    s = s_ref[0] + s_ref[1]
    out_ref[...] = s * dis_ref[...][:, :1]


def _finish(s_parts, dis):
    return pl.pallas_call(
        _finish_body,
        grid=(N // BLK,),
        in_specs=[
            pl.BlockSpec((NC, BLK, D), lambda i: (0, i, 0)),
            pl.BlockSpec((BLK, DEGW), lambda i: (i, 0)),
        ],
        out_specs=pl.BlockSpec((BLK, D), lambda i: (i, 0)),
        out_shape=jax.ShapeDtypeStruct((N, D), jnp.float32),
    )(s_parts, dis)


def kernel(x, edge_index, W1, b1, W2, b2):
    row = jnp.concatenate(
        [edge_index[0].astype(jnp.int32), jnp.zeros((E_PAD - E,), jnp.int32)]
    )
    col = jnp.concatenate(
        [edge_index[1].astype(jnp.int32),
         jnp.full((E_PAD - E,), TRASH, jnp.int32)]
    )
    zeros_d = jnp.zeros((N_PAD, D), jnp.float32)
    ones_rows = jnp.ones((DCH, D), jnp.float32)

    deg_scatter, edge_scatter = _sc_kernels()
    deg_parts = deg_scatter(col, ones_rows, zeros_d)
    g1, dis = _layer1(deg_parts, x, W1, b1.reshape(1, D))
    s1 = edge_scatter(g1, row, col, zeros_d)
    g2 = _layer2(s1, dis, W2, b2.reshape(1, D))
    s2 = edge_scatter(g2, row, col, zeros_d)
    return _finish(s2, dis)


# speedup vs baseline: 1.5507x; 1.5507x over previous
import functools

import jax
import jax.numpy as jnp
from jax import lax
from jax.experimental import pallas as pl
from jax.experimental.pallas import tpu as pltpu
from jax.experimental.pallas import tpu_sc as plsc

N = 10000
E = 320000
D = 128

NC = 2
NS = 16
DEGW = 16
CH = 96
ECT = -(-E // (CH * NS * NC))
E_EDGE = ECT * NS * NC * CH
DCH = 128
DCT = 2 * -(-E // (DCH * NS * NC * 2))
E_PAD = DCT * NS * NC * DCH
TRASH = N
N_PAD = 10112
RPT = N_PAD // NS

@functools.cache
def _sc_kernels():
    mesh = plsc.VectorSubcoreMesh(
        core_axis_name="c", subcore_axis_name="s", num_cores=NC, num_subcores=NS
    )
    deg = functools.partial(
        pl.kernel,
        out_type=jax.ShapeDtypeStruct((NC, N_PAD, D), jnp.float32),
        mesh=mesh,
        scratch_types=[
            pltpu.VMEM((DCH,), jnp.int32),
            pltpu.VMEM((DCH,), jnp.int32),
            pltpu.VMEM((DCH, D), jnp.float32),
            pltpu.SemaphoreType.DMA,
            pltpu.SemaphoreType.DMA,
            pltpu.VMEM_SHARED((N_PAD, D), jnp.float32),
        ],
    )(_deg_scatter_body)
    edge = functools.partial(
        pl.kernel,
        out_type=jax.ShapeDtypeStruct((NC, N_PAD, D), jnp.float32),
        mesh=mesh,
        scratch_types=(
            [pltpu.VMEM((CH,), jnp.int32) for _ in range(4)]
            + [pltpu.VMEM((CH,), jnp.int32) for _ in range(4)]
            + [pltpu.VMEM((CH, D), jnp.float32) for _ in range(4)]
            + [pltpu.SemaphoreType.DMA for _ in range(4)]
            + [pltpu.SemaphoreType.DMA for _ in range(4)]
            + [pltpu.VMEM_SHARED((N_PAD, D), jnp.float32)]
        ),
    )(_edge_scatter_body)
    return deg, edge


def _tile_chunk0(c, s, cpt):
    return (c * NS + s) * cpt


def _deg_scatter_body(col_hbm, ones_hbm, zeros_hbm, out_hbm,
                      cidx0, cidx1, ones_v, sem0, sem1, acc):
    c = lax.axis_index("c")
    s = lax.axis_index("s")
    rbase = pl.multiple_of(s * RPT, 8)
    pltpu.sync_copy(zeros_hbm.at[pl.ds(rbase, RPT)], acc.at[pl.ds(rbase, RPT)])
    pltpu.sync_copy(ones_hbm, ones_v)
    plsc.subcore_barrier()
    chunk0 = _tile_chunk0(c, s, DCT)
    m = DCT // 2

    def _eoff(q):
        return pl.multiple_of((chunk0 + q) * DCH, 8)

    pltpu.sync_copy(col_hbm.at[pl.ds(_eoff(0), DCH)], cidx0)

    def pair(p, carry):
        qa = 2 * p
        pltpu.async_copy(ones_v, acc.at[cidx0], sem0, add=True)
        pltpu.sync_copy(col_hbm.at[pl.ds(_eoff(qa + 1), DCH)], cidx1)
        pltpu.async_copy(ones_v, acc.at[cidx1], sem1, add=True)
        pltpu.make_async_copy(ones_v, acc.at[cidx0], sem0).wait()
        qn = jnp.minimum(qa + 2, DCT - 1)
        pltpu.sync_copy(col_hbm.at[pl.ds(_eoff(qn), DCH)], cidx0)
        pltpu.make_async_copy(ones_v, acc.at[cidx1], sem1).wait()
        return carry

    lax.fori_loop(0, m, pair, 0)
    plsc.subcore_barrier()
    pltpu.sync_copy(acc.at[pl.ds(rbase, RPT)], out_hbm.at[c, pl.ds(rbase, RPT)])


def _edge_scatter_body(g_hbm, row_hbm, col_hbm, zeros_hbm, out_hbm,
                       ridx0, ridx1, ridx2, ridx3,
                       cidx0, cidx1, cidx2, cidx3,
                       rows0, rows1, rows2, rows3,
                       semg0, semg1, semg2, semg3,
                       sems0, sems1, sems2, sems3, acc):
    c = lax.axis_index("c")
    s = lax.axis_index("s")
    rbase = pl.multiple_of(s * RPT, 8)
    pltpu.sync_copy(zeros_hbm.at[pl.ds(rbase, RPT)], acc.at[pl.ds(rbase, RPT)])
    plsc.subcore_barrier()
    chunk0 = _tile_chunk0(c, s, ECT)
    n = ECT

    ridx = (ridx0, ridx1, ridx2, ridx3)
    cidx = (cidx0, cidx1, cidx2, cidx3)
    rows = (rows0, rows1, rows2, rows3)
    semg = (semg0, semg1, semg2, semg3)
    sems = (sems0, sems1, sems2, sems3)

    def _eoff(q):
        return pl.multiple_of((chunk0 + q) * CH, 8)

    def _load_idx(q, b):
        off = _eoff(q)
        pltpu.sync_copy(row_hbm.at[pl.ds(off, CH)], ridx[b])
        pltpu.sync_copy(col_hbm.at[pl.ds(off, CH)], cidx[b])

    def _gather(b):
        pltpu.async_copy(g_hbm.at[ridx[b]], rows[b], semg[b])

    def _gather_wait(b):
        pltpu.make_async_copy(g_hbm.at[ridx[b]], rows[b], semg[b]).wait()

    def _scatter(b):
        pltpu.async_copy(rows[b], acc.at[cidx[b]], sems[b], add=True)

    def _scatter_wait(b):
        pltpu.make_async_copy(rows[b], acc.at[cidx[b]], sems[b]).wait()

    _load_idx(0, 0)
    _gather(0)
    _load_idx(1, 1)
    _gather(1)
    _gather_wait(0)
    _scatter(0)
    _load_idx(2, 2)
    _gather(2)
    _gather_wait(1)
    _scatter(1)
    _load_idx(3, 3)
    _gather(3)

    m = (n - 2) // 4

    def quad(p, carry):
        q = 2 + 4 * p
        for k, b in enumerate((2, 3, 0, 1)):
            _gather_wait(b)
            _scatter_wait((b + 2) % 4)
            _scatter(b)
            nb = (b + 2) % 4
            _load_idx(jnp.minimum(q + k + 2, n - 1), nb)
            _gather(nb)
        return carry

    lax.fori_loop(0, m, quad, 0)

    _scatter_wait(0)
    _scatter_wait(1)
    _gather_wait(2)
    _gather_wait(3)

    def tail(q, carry):
        _load_idx(q, 2)
        _gather(2)
        _gather_wait(2)
        _scatter(2)
        _scatter_wait(2)
        return carry

    lax.fori_loop(2 + 4 * m, n, tail, 0)
    plsc.subcore_barrier()
    pltpu.sync_copy(acc.at[pl.ds(rbase, RPT)], out_hbm.at[c, pl.ds(rbase, RPT)])


BLK = 2000


def _layer1_body(deg_ref, x_ref, w_ref, b_ref, g_ref, dis_ref):
    deg = deg_ref[0] + deg_ref[1]
    dis = jnp.where(deg > 0.0, lax.rsqrt(deg), 0.0)
    h = lax.dot_general(
        x_ref[...], w_ref[...], (((1,), (1,)), ((), ())),
        preferred_element_type=jnp.float32,
    ) + b_ref[...]
    g_ref[...] = h * dis[:, :1]
    dis_ref[...] = dis[:, :DEGW]


def _layer1(deg_parts, x, W1, b1):
    return pl.pallas_call(
        _layer1_body,
        grid=(N // BLK,),
        in_specs=[
            pl.BlockSpec((NC, BLK, D), lambda i: (0, i, 0)),
            pl.BlockSpec((BLK, D), lambda i: (i, 0)),
            pl.BlockSpec((D, D), lambda i: (0, 0)),
            pl.BlockSpec((1, D), lambda i: (0, 0)),
        ],
        out_specs=[
            pl.BlockSpec((BLK, D), lambda i: (i, 0)),
            pl.BlockSpec((BLK, DEGW), lambda i: (i, 0)),
        ],
        out_shape=[
            jax.ShapeDtypeStruct((N, D), jnp.float32),
            jax.ShapeDtypeStruct((N, DEGW), jnp.float32),
        ],
    )(deg_parts, x, W1, b1)


def _layer2_body(s_ref, dis_ref, w_ref, b_ref, g_ref):
    s = s_ref[0] + s_ref[1]
    out1 = s * dis_ref[...][:, :1]
    t = jnp.maximum(out1, 0.0)
    h = lax.dot_general(
        t, w_ref[...], (((1,), (1,)), ((), ())),
        preferred_element_type=jnp.float32,
    ) + b_ref[...]
    g_ref[...] = h * dis_ref[...][:, :1]


def _layer2(s_parts, dis, W2, b2):
    return pl.pallas_call(
        _layer2_body,
        grid=(N // BLK,),
        in_specs=[
            pl.BlockSpec((NC, BLK, D), lambda i: (0, i, 0)),
            pl.BlockSpec((BLK, DEGW), lambda i: (i, 0)),
            pl.BlockSpec((D, D), lambda i: (0, 0)),
            pl.BlockSpec((1, D), lambda i: (0, 0)),
        ],
        out_specs=pl.BlockSpec((BLK, D), lambda i: (i, 0)),
        out_shape=jax.ShapeDtypeStruct((N, D), jnp.float32),
    )(s_parts, dis, W2, b2)


def _finish_body(s_ref, dis_ref, out_ref):
    s = s_ref[0] + s_ref[1]
    out_ref[...] = s * dis_ref[...][:, :1]


def _finish(s_parts, dis):
    return pl.pallas_call(
        _finish_body,
        grid=(N // BLK,),
        in_specs=[
            pl.BlockSpec((NC, BLK, D), lambda i: (0, i, 0)),
            pl.BlockSpec((BLK, DEGW), lambda i: (i, 0)),
        ],
        out_specs=pl.BlockSpec((BLK, D), lambda i: (i, 0)),
        out_shape=jax.ShapeDtypeStruct((N, D), jnp.float32),
    )(s_parts, dis)


def kernel(x, edge_index, W1, b1, W2, b2):
    npad = E_PAD - E
    row = jnp.concatenate(
        [edge_index[0].astype(jnp.int32),
         jnp.arange(npad, dtype=jnp.int32) % N]
    )
    col = jnp.concatenate(
        [edge_index[1].astype(jnp.int32),
         TRASH + jnp.arange(npad, dtype=jnp.int32) % (N_PAD - TRASH)]
    )
    zeros_d = jnp.zeros((N_PAD, D), jnp.float32)
    ones_rows = jnp.ones((DCH, D), jnp.float32)

    deg_scatter, edge_scatter = _sc_kernels()
    deg_parts = deg_scatter(col, ones_rows, zeros_d)
    g1, dis = _layer1(deg_parts, x, W1, b1.reshape(1, D))
    s1 = edge_scatter(g1, row, col, zeros_d)
    g2 = _layer2(s1, dis, W2, b2.reshape(1, D))
    s2 = edge_scatter(g2, row, col, zeros_d)
    return _finish(s2, dis)
